# Initial kernel scaffold; baseline (speedup 1.0000x reference)
import functools
import jax
import jax.numpy as jnp
from jax.experimental import pallas as pl
from jax.experimental.pallas import tpu as pltpu

ROWS = 8192
COLS = 128
GRID = 16
BLK = ROWS // GRID


def _tc_kernel(x_ref, lbl_ref, out_ref, bits_ref, *, keep, nblocks):
    step = pl.program_id(0)
    x0 = x_ref[0]
    x1 = x_ref[1]
    lbl = lbl_ref[...]
    diff = x1 - x0
    d = jnp.where(lbl == 0, diff, -diff)
    pe = jnp.maximum(d, 0.0) + jnp.log1p(jnp.exp(-jnp.abs(d)))
    pe = jnp.where(lbl < 0, 0.0, pe)
    bits_ref[pl.ds(step * BLK, BLK), :] = jax.lax.bitcast_convert_type(
        pe, jnp.int32)

    @pl.when(step == nblocks - 1)
    def _():
        bits = bits_ref[...]

        def body(_, carry):
            lo, hi = carry
            mid = lo + (hi - lo) // 2
            cnt = jnp.sum((bits > mid).astype(jnp.int32))
            big = cnt >= keep
            return jnp.where(big, mid, lo), jnp.where(big, hi, mid)

        lo, hi = jax.lax.fori_loop(
            0, 31, body, (jnp.int32(-1), jnp.int32(2147483647)))
        tval = jax.lax.bitcast_convert_type(hi, jnp.float32)
        pe_all = jax.lax.bitcast_convert_type(bits, jnp.float32)
        gt = bits > hi
        sum_gt = jnp.sum(jnp.where(gt, pe_all, 0.0))
        cnt_gt = jnp.sum(gt.astype(jnp.int32))
        res = (sum_gt + (keep - cnt_gt).astype(jnp.float32) * tval) / keep
        out_ref[0, 0] = res


def kernel(class_out, label):
    n = label.shape[0]
    keep = int(n * 0.7)
    xt = jnp.transpose(class_out.astype(jnp.float32)).reshape(2, ROWS, COLS)
    lbl = label.astype(jnp.int32).reshape(ROWS, COLS)
    out = pl.pallas_call(
        functools.partial(_tc_kernel, keep=keep, nblocks=GRID),
        grid=(GRID,),
        in_specs=[
            pl.BlockSpec((2, BLK, COLS), lambda i: (0, i, 0)),
            pl.BlockSpec((BLK, COLS), lambda i: (i, 0)),
        ],
        out_specs=pl.BlockSpec(
            (1, 1), lambda i: (0, 0), memory_space=pltpu.SMEM),
        out_shape=jax.ShapeDtypeStruct((1, 1), jnp.float32),
        scratch_shapes=[pltpu.VMEM((ROWS, COLS), jnp.int32)],
    )(xt, lbl)
    return out[0, 0]


# speedup vs baseline: 15.2663x; 15.2663x over previous
import functools
import jax
import jax.numpy as jnp
from jax.experimental import pallas as pl
from jax.experimental.pallas import tpu as pltpu

ROWS = 8192
COLS = 128
GRID = 16
BLK = ROWS // GRID


def _tc_kernel(x_ref, lbl_ref, out_ref, bits_ref, *, keep, nblocks):
    step = pl.program_id(0)
    x0 = x_ref[0]
    x1 = x_ref[1]
    lbl = lbl_ref[...]
    diff = x1 - x0
    d = jnp.where(lbl == 0, diff, -diff)
    pe = jnp.maximum(d, 0.0) + jnp.log1p(jnp.exp(-jnp.abs(d)))
    pe = jnp.where(lbl < 0, 0.0, pe)
    bits_ref[pl.ds(step * BLK, BLK), :] = jax.lax.bitcast_convert_type(
        pe, jnp.int32)

    @pl.when(step == nblocks - 1)
    def _():
        bits = bits_ref[...]

        def body(i, t):
            cand = t | (jnp.int32(1) << (jnp.int32(30) - i))
            cnt = jnp.sum((bits >= cand).astype(jnp.int32))
            return jnp.where(cnt >= keep, cand, t)

        t = jax.lax.fori_loop(0, 31, body, jnp.int32(0))
        tval = jax.lax.bitcast_convert_type(t, jnp.float32)
        pe_all = jax.lax.bitcast_convert_type(bits, jnp.float32)
        gt = bits > t
        sum_gt = jnp.sum(jnp.where(gt, pe_all, 0.0))
        cnt_gt = jnp.sum(gt.astype(jnp.int32))
        res = (sum_gt + (keep - cnt_gt).astype(jnp.float32) * tval) / keep
        out_ref[0, 0] = res


def kernel(class_out, label):
    n = label.shape[0]
    keep = int(n * 0.7)
    xt = jnp.transpose(class_out.astype(jnp.float32)).reshape(2, ROWS, COLS)
    lbl = label.astype(jnp.int32).reshape(ROWS, COLS)
    out = pl.pallas_call(
        functools.partial(_tc_kernel, keep=keep, nblocks=GRID),
        grid=(GRID,),
        in_specs=[
            pl.BlockSpec((2, BLK, COLS), lambda i: (0, i, 0)),
            pl.BlockSpec((BLK, COLS), lambda i: (i, 0)),
        ],
        out_specs=pl.BlockSpec(
            (1, 1), lambda i: (0, 0), memory_space=pltpu.SMEM),
        out_shape=jax.ShapeDtypeStruct((1, 1), jnp.float32),
        scratch_shapes=[pltpu.VMEM((ROWS, COLS), jnp.int32)],
    )(xt, lbl)
    return out[0, 0]
